# baseline (speedup 1.0000x reference)
import functools

import jax
import jax.numpy as jnp
from jax import lax
from jax.experimental import pallas as pl
from jax.experimental.pallas import tpu as pltpu
from jax.experimental.pallas import tpu_sc as plsc

VOCAB = 100000
D = 300
DPH = 320
DW = DPH // 2
HIST = 50
RPG = 6
IPG = RPG * HIST
GL = IPG + 4
B = 4096
BT = 3 * B
NC = 2
NS = 16
NW = NC * NS
ROWS_PER_W = BT // NW
GROUPS_PER_W = ROWS_PER_W // RPG
NGRP = BT // RPG
GROUP = 24
LANES = 16
NCHUNK = DW // LANES

_mesh = plsc.VectorSubcoreMesh(core_axis_name="c", subcore_axis_name="s")


@functools.partial(
    pl.kernel,
    mesh=_mesh,
    out_type=jax.ShapeDtypeStruct((BT, DW), jnp.int32),
    scratch_types=[
        pltpu.VMEM((GROUPS_PER_W, GL), jnp.int32),
        pltpu.VMEM((GL, DW), jnp.int32),
        pltpu.VMEM((GL, DW), jnp.int32),
        pltpu.VMEM((GROUP, DW), jnp.int32),
        pltpu.SemaphoreType.DMA,
        pltpu.SemaphoreType.DMA,
    ],
    compiler_params=pltpu.CompilerParams(
        use_tc_tiling_on_sc=False, needs_layout_passes=False
    ),
)
def _pool_kernel(
    idx_hbm, emb_hbm, out_hbm, idx_v, rows0, rows1, outb, sem0, sem1
):
    wid = lax.axis_index("s") * NC + lax.axis_index("c")
    base = pl.multiple_of(wid * ROWS_PER_W, ROWS_PER_W)
    gbase = pl.multiple_of(wid * GROUPS_PER_W, GROUPS_PER_W)
    pltpu.sync_copy(idx_hbm.at[pl.ds(gbase, GROUPS_PER_W)], idx_v)

    bufs = (rows0, rows1)
    sems = (sem0, sem1)
    NBUF = 2

    for i in range(NBUF):
        pltpu.async_copy(emb_hbm.at[idx_v.at[i]], bufs[i], sems[i])

    scale = jnp.float32(1.0 / HIST)

    def ring_body(it, _):
        for bi in range(NBUF):
            g = it * NBUF + bi
            buf = bufs[bi]
            sem = sems[bi]
            pltpu.make_async_copy(emb_hbm.at[idx_v.at[g]], buf, sem).wait()

            slot = lax.rem(RPG * g, GROUP)

            def chunk_body(j, _):
                off = j * LANES

                def halves(row0):
                    pe = [jnp.zeros((LANES,), jnp.float32) for _ in range(2)]
                    po = [jnp.zeros((LANES,), jnp.float32) for _ in range(2)]
                    for r in range(HIST):
                        w = buf[row0 + r, pl.ds(off, LANES)]
                        e, o = plsc.unpack(
                            plsc.bitcast(w, jnp.bfloat16),
                            format=plsc.PackFormat.INTERLEAVED,
                        )
                        pe[r % 2] = pe[r % 2] + e
                        po[r % 2] = po[r % 2] + o
                    return (pe[0] + pe[1]) * scale, (po[0] + po[1]) * scale

                for sub in range(RPG):
                    ae, ao = halves(sub * HIST)
                    outb[slot + sub, pl.ds(off, LANES)] = plsc.bitcast(
                        plsc.pack(ae, ao, format=plsc.PackFormat.INTERLEAVED),
                        jnp.int32,
                    )
                return 0

            lax.fori_loop(0, NCHUNK, chunk_body, 0)

            @pl.when(g + NBUF < GROUPS_PER_W)
            def _():
                pltpu.async_copy(emb_hbm.at[idx_v.at[g + NBUF]], buf, sem)

            @pl.when(slot == GROUP - RPG)
            def _():
                flush_base = pl.multiple_of(
                    base + RPG * g - (GROUP - RPG), GROUP
                )
                pltpu.sync_copy(outb, out_hbm.at[pl.ds(flush_base, GROUP)])
        return 0

    lax.fori_loop(0, GROUPS_PER_W // NBUF, ring_body, 0)


def _proj_body(x_ref, w_ref, b_ref, o_ref):
    o_ref[...] = (
        lax.dot_general(
            x_ref[...],
            w_ref[...],
            (((1,), (1,)), ((), ())),
            preferred_element_type=jnp.float32,
        )
        + b_ref[...]
    )


_BLK = 1024


def _proj(pooled, W_pad, b2):
    return pl.pallas_call(
        _proj_body,
        grid=(BT // _BLK,),
        in_specs=[
            pl.BlockSpec((_BLK, DPH), lambda i: (i, 0)),
            pl.BlockSpec((D, DPH), lambda i: (0, 0)),
            pl.BlockSpec((1, D), lambda i: (0, 0)),
        ],
        out_specs=pl.BlockSpec((_BLK, D), lambda i: (i, 0)),
        out_shape=jax.ShapeDtypeStruct((BT, D), jnp.float32),
    )(pooled, W_pad, b2)


@jax.jit
def kernel(q, p, n, emb, W, b):
    idx_all = jnp.concatenate(
        [q.astype(jnp.int32), p.astype(jnp.int32), n.astype(jnp.int32)], axis=0
    )
    idx_pack = jnp.pad(idx_all.reshape(NGRP, IPG), ((0, 0), (0, GL - IPG)))
    emb_i32 = lax.bitcast_convert_type(
        emb.astype(jnp.bfloat16).reshape(VOCAB, D // 2, 2), jnp.int32
    )
    emb_i32 = jnp.pad(emb_i32, ((0, 0), (0, DW - D // 2)))
    pooled_i32 = _pool_kernel(idx_pack, emb_i32)
    pooled_bf = lax.bitcast_convert_type(pooled_i32, jnp.bfloat16).reshape(
        BT, DPH
    )
    W_pad = jnp.pad(W, ((0, 0), (0, DPH - D)))
    out = _proj(pooled_bf, W_pad, b.reshape(1, D))
    return (out[:B], out[B : 2 * B], out[2 * B :])


# speedup vs baseline: 1.1953x; 1.1953x over previous
import functools

import jax
import jax.numpy as jnp
from jax import lax
from jax.experimental import pallas as pl
from jax.experimental.pallas import tpu as pltpu
from jax.experimental.pallas import tpu_sc as plsc

VOCAB = 100000
D = 300
DPH = 320
DW = DPH // 2
HIST = 50
RPG = 4
GL = RPG * HIST
B = 4096
BT = 3 * B
NC = 2
NS = 16
NW = NC * NS
ROWS_PER_W = BT // NW
GROUPS_PER_W = ROWS_PER_W // RPG
NGRP = BT // RPG
GROUP = 32
LANES = 16
NCHUNK = DW // LANES

_mesh = plsc.VectorSubcoreMesh(core_axis_name="c", subcore_axis_name="s")


@functools.partial(
    pl.kernel,
    mesh=_mesh,
    out_type=jax.ShapeDtypeStruct((BT, DW), jnp.int32),
    scratch_types=[
        pltpu.VMEM((GROUPS_PER_W, GL), jnp.int32),
        pltpu.VMEM((GL, DW), jnp.int32),
        pltpu.VMEM((GL, DW), jnp.int32),
        pltpu.VMEM((GL, DW), jnp.int32),
        pltpu.VMEM((GROUP, DW), jnp.int32),
        pltpu.SemaphoreType.DMA,
        pltpu.SemaphoreType.DMA,
        pltpu.SemaphoreType.DMA,
    ],
    compiler_params=pltpu.CompilerParams(
        use_tc_tiling_on_sc=False, needs_layout_passes=False
    ),
)
def _pool_kernel(
    idx_hbm, emb_hbm, out_hbm, idx_v, rows0, rows1, rows2, outb, sem0, sem1, sem2
):
    wid = lax.axis_index("s") * NC + lax.axis_index("c")
    base = pl.multiple_of(wid * ROWS_PER_W, ROWS_PER_W)
    gbase = pl.multiple_of(wid * GROUPS_PER_W, GROUPS_PER_W)
    pltpu.sync_copy(idx_hbm.at[pl.ds(gbase, GROUPS_PER_W)], idx_v)

    bufs = (rows0, rows1, rows2)
    sems = (sem0, sem1, sem2)
    NBUF = 3

    for i in range(NBUF):
        pltpu.async_copy(emb_hbm.at[idx_v.at[i]], bufs[i], sems[i])

    scale = jnp.float32(1.0 / HIST)

    def ring_body(it, _):
        for bi in range(NBUF):
            g = it * NBUF + bi
            buf = bufs[bi]
            sem = sems[bi]
            pltpu.make_async_copy(emb_hbm.at[idx_v.at[g]], buf, sem).wait()

            slot = lax.rem(RPG * g, GROUP)

            def chunk_body(j, _):
                off = j * LANES

                def halves(row0):
                    pe = [jnp.zeros((LANES,), jnp.float32) for _ in range(2)]
                    po = [jnp.zeros((LANES,), jnp.float32) for _ in range(2)]
                    for r in range(HIST):
                        w = buf[row0 + r, pl.ds(off, LANES)]
                        e, o = plsc.unpack(
                            plsc.bitcast(w, jnp.bfloat16),
                            format=plsc.PackFormat.INTERLEAVED,
                        )
                        pe[r % 2] = pe[r % 2] + e
                        po[r % 2] = po[r % 2] + o
                    return (pe[0] + pe[1]) * scale, (po[0] + po[1]) * scale

                for sub in range(RPG):
                    ae, ao = halves(sub * HIST)
                    outb[slot + sub, pl.ds(off, LANES)] = plsc.bitcast(
                        plsc.pack(ae, ao, format=plsc.PackFormat.INTERLEAVED),
                        jnp.int32,
                    )
                return 0

            lax.fori_loop(0, NCHUNK, chunk_body, 0)

            @pl.when(g + NBUF < GROUPS_PER_W)
            def _():
                pltpu.async_copy(emb_hbm.at[idx_v.at[g + NBUF]], buf, sem)

            @pl.when(slot == GROUP - RPG)
            def _():
                flush_base = pl.multiple_of(
                    base + RPG * g - (GROUP - RPG), GROUP
                )
                pltpu.sync_copy(outb, out_hbm.at[pl.ds(flush_base, GROUP)])
        return 0

    lax.fori_loop(0, GROUPS_PER_W // NBUF, ring_body, 0)


def _proj_body(x_ref, w_ref, b_ref, o_ref):
    o_ref[...] = (
        lax.dot_general(
            x_ref[...],
            w_ref[...],
            (((1,), (1,)), ((), ())),
            preferred_element_type=jnp.float32,
        )
        + b_ref[...]
    )


_BLK = 1024


def _proj(pooled, W_pad, b2):
    return pl.pallas_call(
        _proj_body,
        grid=(BT // _BLK,),
        in_specs=[
            pl.BlockSpec((_BLK, DPH), lambda i: (i, 0)),
            pl.BlockSpec((D, DPH), lambda i: (0, 0)),
            pl.BlockSpec((1, D), lambda i: (0, 0)),
        ],
        out_specs=pl.BlockSpec((_BLK, D), lambda i: (i, 0)),
        out_shape=jax.ShapeDtypeStruct((BT, D), jnp.float32),
    )(pooled, W_pad, b2)


@jax.jit
def kernel(q, p, n, emb, W, b):
    idx_all = jnp.concatenate(
        [q.astype(jnp.int32), p.astype(jnp.int32), n.astype(jnp.int32)], axis=0
    )
    idx_pack = idx_all.reshape(NGRP, GL)
    emb_i32 = lax.bitcast_convert_type(
        emb.astype(jnp.bfloat16).reshape(VOCAB, D // 2, 2), jnp.int32
    )
    emb_i32 = jnp.pad(emb_i32, ((0, 0), (0, DW - D // 2)))
    pooled_i32 = _pool_kernel(idx_pack, emb_i32)
    pooled_bf = lax.bitcast_convert_type(pooled_i32, jnp.bfloat16).reshape(
        BT, DPH
    )
    W_pad = jnp.pad(W, ((0, 0), (0, DPH - D)))
    out = _proj(pooled_bf, W_pad, b.reshape(1, D))
    return (out[:B], out[B : 2 * B], out[2 * B :])
